# baseline (speedup 1.0000x reference)
import functools

import jax
import jax.numpy as jnp
from jax import lax
from jax.experimental import pallas as pl
from jax.experimental.pallas import tpu as pltpu
from jax.experimental.pallas import tpu_sc as plsc

_N = 10000
_E = 320000
_E8 = _E // 8
_E16 = _E * 16
_DN = 128
_DE = 16
_DG = 32
_B = 16
_NMP = 3
_EPS = 1e-5

_NC = 2
_NS = 16
_NW = _NC * _NS
_EW = _E // _NW
_CHUNK = 720
_CHUNKS = [(i * 720, 720) for i in range(13)] + [(9360, 640)]
_NCHK = len(_CHUNKS)
_NP = 10240
_RPT = _NP // _NS


def _stats_body(x_ref, s_ref, ss_ref):
    @pl.when(pl.program_id(0) == 0)
    def _init():
        s_ref[...] = jnp.zeros_like(s_ref)
        ss_ref[...] = jnp.zeros_like(ss_ref)

    xb = x_ref[...]
    s_ref[...] += jnp.sum(xb, axis=0, keepdims=True)
    ss_ref[...] += jnp.sum(xb * xb, axis=0, keepdims=True)


def _colstats(x, bn):
    r, d = x.shape
    return pl.pallas_call(
        _stats_body,
        grid=(r // bn,),
        in_specs=[pl.BlockSpec((bn, d), lambda i: (i, 0))],
        out_specs=[pl.BlockSpec((1, d), lambda i: (0, 0))] * 2,
        out_shape=[jax.ShapeDtypeStruct((1, d), jnp.float32)] * 2,
    )(x)


def _proj_body(x_ref, ws_ref, wd_ref, ue_ref, bc_ref, xs_ref, xd_ref):
    xb = x_ref[...]
    bn = xb.shape[0]
    iot = lax.broadcasted_iota(jnp.int32, (bn, _B), 1)
    oh = (bc_ref[...] == iot).astype(jnp.float32)
    xs_ref[...] = (
        jnp.dot(xb, ws_ref[...], preferred_element_type=jnp.float32)
        + jnp.dot(oh, ue_ref[...], preferred_element_type=jnp.float32)
    )
    xd_ref[...] = jnp.dot(xb, wd_ref[...], preferred_element_type=jnp.float32)


def _proj(x, ws, wd, ueff, bcol, bn=2000):
    return pl.pallas_call(
        _proj_body,
        grid=(_N // bn,),
        in_specs=[
            pl.BlockSpec((bn, _DN), lambda i: (i, 0)),
            pl.BlockSpec((_DN, _DE), lambda i: (0, 0)),
            pl.BlockSpec((_DN, _DE), lambda i: (0, 0)),
            pl.BlockSpec((_B, _DE), lambda i: (0, 0)),
            pl.BlockSpec((bn, 1), lambda i: (i, 0)),
        ],
        out_specs=[pl.BlockSpec((bn, _DE), lambda i: (i, 0))] * 2,
        out_shape=[jax.ShapeDtypeStruct((_N, _DE), jnp.float32)] * 2,
    )(x, ws, wd, ueff, bcol)


def _ep_body(e_ref, w_ref, ep_ref):
    ep_ref[...] = jnp.dot(e_ref[...], w_ref[...],
                          preferred_element_type=jnp.float32)


def _ep8(e8, wbig, be=8000):
    return pl.pallas_call(
        _ep_body,
        grid=(_E8 // be,),
        in_specs=[
            pl.BlockSpec((be, 128), lambda i: (i, 0)),
            pl.BlockSpec((128, 128), lambda i: (0, 0)),
        ],
        out_specs=pl.BlockSpec((be, 128), lambda i: (i, 0)),
        out_shape=jax.ShapeDtypeStruct((_E8, 128), jnp.float32),
    )(e8, wbig)


def _node_body(x_ref, a0_ref, a1_ref, c0_ref, c1_ref, bc_ref, br_ref,
               wnx_ref, wne_ref, un_ref, xn_ref, nagg_ref, ncnt_ref):
    bn = x_ref.shape[0]
    cnt = jnp.maximum(c0_ref[...] + c1_ref[...], 1.0)
    eagg = (a0_ref[...] + a1_ref[...]) / cnt
    iot = lax.broadcasted_iota(jnp.int32, (bn, _B), 1)
    oh = (bc_ref[...] == iot).astype(jnp.float32)
    xn = (
        jnp.dot(x_ref[...], wnx_ref[...], preferred_element_type=jnp.float32)
        + jnp.dot(eagg, wne_ref[...], preferred_element_type=jnp.float32)
        + jnp.dot(oh, un_ref[...], preferred_element_type=jnp.float32)
    )
    xn = jnp.maximum(xn, 0.0)
    xn_ref[...] = xn

    iot_t = lax.broadcasted_iota(jnp.int32, (_B, bn), 0)
    oh_t = (br_ref[...].reshape(1, bn) == iot_t).astype(jnp.float32)

    @pl.when(pl.program_id(0) == 0)
    def _init():
        nagg_ref[...] = jnp.zeros_like(nagg_ref)
        ncnt_ref[...] = jnp.zeros_like(ncnt_ref)

    nagg_ref[...] += jnp.dot(oh_t, xn, preferred_element_type=jnp.float32)
    ncnt_ref[...] += jnp.sum(oh_t, axis=1, keepdims=True)


def _node(x, a0, a1, c0, c1, bcol, brow, wnx, wne, unrow, bn=2000):
    return pl.pallas_call(
        _node_body,
        grid=(_N // bn,),
        in_specs=[
            pl.BlockSpec((bn, _DN), lambda i: (i, 0)),
            pl.BlockSpec((bn, _DE), lambda i: (i, 0)),
            pl.BlockSpec((bn, _DE), lambda i: (i, 0)),
            pl.BlockSpec((bn, 1), lambda i: (i, 0)),
            pl.BlockSpec((bn, 1), lambda i: (i, 0)),
            pl.BlockSpec((bn, 1), lambda i: (i, 0)),
            pl.BlockSpec((1, 1, bn), lambda i: (i, 0, 0)),
            pl.BlockSpec((_DN, _DN), lambda i: (0, 0)),
            pl.BlockSpec((_DE, _DN), lambda i: (0, 0)),
            pl.BlockSpec((_B, _DN), lambda i: (0, 0)),
        ],
        out_specs=[
            pl.BlockSpec((bn, _DN), lambda i: (i, 0)),
            pl.BlockSpec((_B, _DN), lambda i: (0, 0)),
            pl.BlockSpec((_B, 1), lambda i: (0, 0)),
        ],
        out_shape=[
            jax.ShapeDtypeStruct((_N, _DN), jnp.float32),
            jax.ShapeDtypeStruct((_B, _DN), jnp.float32),
            jax.ShapeDtypeStruct((_B, 1), jnp.float32),
        ],
    )(x, a0, a1, c0, c1, bcol, brow, wnx, wne, unrow)


def _glob_body(has_next, nagg_ref, ncnt_ref, u_ref, wgx_ref, wgu_ref, bg_ref,
               *rest):
    if has_next:
        weu_ref, be_ref, wnu_ref, bnn_ref, un_ref, ueff_ref, unrow_ref = rest
    else:
        (un_ref,) = rest
    nagg = nagg_ref[...] / jnp.maximum(ncnt_ref[...], 1.0)
    un = (
        jnp.dot(nagg, wgx_ref[...], preferred_element_type=jnp.float32)
        + jnp.dot(u_ref[...], wgu_ref[...], preferred_element_type=jnp.float32)
        + bg_ref[...]
    )
    un = jnp.maximum(un, 0.0)
    un_ref[...] = un
    if has_next:
        ueff_ref[...] = (
            jnp.dot(un, weu_ref[...], preferred_element_type=jnp.float32)
            + be_ref[...]
        )
        unrow_ref[...] = (
            jnp.dot(un, wnu_ref[...], preferred_element_type=jnp.float32)
            + bnn_ref[...]
        )


def _glob(has_next, naggsum, ncnt, u, wgx, wgu, bg,
          weu=None, be=None, wnu=None, bnn=None):
    out_shape = [jax.ShapeDtypeStruct((_B, _DG), jnp.float32)]
    args = [naggsum, ncnt, u, wgx, wgu, bg]
    if has_next:
        out_shape += [
            jax.ShapeDtypeStruct((_B, _DE), jnp.float32),
            jax.ShapeDtypeStruct((_B, _DN), jnp.float32),
        ]
        args += [weu, be, wnu, bnn]
    return pl.pallas_call(
        functools.partial(_glob_body, has_next),
        out_shape=out_shape,
    )(*args)


def _glob0_body(g_ref, gg_ref, gb_ref, tx_ref, te_ref,
                wes_ref, wed_ref, wee_ref, weu_ref, be_ref,
                wnx_ref, wnu_ref, bn_ref,
                u0_ref, ueff_ref, unrow_ref):
    g = g_ref[...]
    m = jnp.mean(g, axis=0, keepdims=True)
    v = jnp.mean(g * g, axis=0, keepdims=True) - m * m
    ub = (g - m) * lax.rsqrt(v + _EPS) * gg_ref[...] + gb_ref[...]
    u0_ref[...] = ub
    tx = tx_ref[...]
    te = te_ref[...]
    ueff_ref[...] = (
        jnp.dot(ub, weu_ref[...], preferred_element_type=jnp.float32)
        + be_ref[...]
        + jnp.dot(te, wee_ref[...], preferred_element_type=jnp.float32)
        + jnp.dot(tx, wes_ref[...], preferred_element_type=jnp.float32)
        + jnp.dot(tx, wed_ref[...], preferred_element_type=jnp.float32)
    )
    unrow_ref[...] = (
        jnp.dot(ub, wnu_ref[...], preferred_element_type=jnp.float32)
        + bn_ref[...]
        + jnp.dot(tx, wnx_ref[...], preferred_element_type=jnp.float32)
    )


def _glob0(g, gg, gb, tx, te, wes, wed, wee, weu, be, wnx, wnu, bnn):
    return pl.pallas_call(
        _glob0_body,
        out_shape=[
            jax.ShapeDtypeStruct((_B, _DG), jnp.float32),
            jax.ShapeDtypeStruct((_B, _DE), jnp.float32),
            jax.ShapeDtypeStruct((_B, _DN), jnp.float32),
        ],
    )(g, gg, gb, tx, te, wes, wed, wee, weu, be, wnx, wnu, bnn)


_MESH = dict(core_axis_name="c", subcore_axis_name="s",
             num_cores=_NC, num_subcores=_NS)


def _sc_prep_body(dst_hbm, cnt_hbm, dst_v, dst_s, ones_v, cnt_sh):
    c = lax.axis_index("c")
    s = lax.axis_index("s")
    w = c * _NS + s

    def zrow(i, _):
        ones_v[i, :] = jnp.zeros((16,), jnp.float32)
        return 0
    lax.fori_loop(0, _RPT, zrow, 0)
    pltpu.sync_copy(ones_v.at[pl.ds(0, _RPT)], cnt_sh.at[pl.ds(s * _RPT, _RPT)])
    plsc.subcore_barrier()

    def orow(i, _):
        ones_v[i, :] = jnp.ones((16,), jnp.float32)
        return 0
    lax.fori_loop(0, _CHUNK, orow, 0)

    for off, sz in _CHUNKS:
        base = w * _EW + off
        dv = dst_v if sz == _CHUNK else dst_s
        pltpu.sync_copy(dst_hbm.at[pl.ds(base, sz)], dv)
        pltpu.sync_copy(ones_v.at[pl.ds(0, sz)], cnt_sh.at[dv], add=True)

    plsc.subcore_barrier()
    pltpu.sync_copy(cnt_sh.at[pl.ds(s * _RPT, _RPT)],
                    cnt_hbm.at[c, pl.ds(s * _RPT, _RPT)])


def _sc_prep(dst):
    return pl.kernel(
        _sc_prep_body,
        out_type=jax.ShapeDtypeStruct((_NC, _NP, _DE), jnp.float32),
        mesh=plsc.VectorSubcoreMesh(**_MESH),
        compiler_params=pltpu.CompilerParams(use_tc_tiling_on_sc=False),
        scratch_types=[
            pltpu.VMEM((_CHUNK,), jnp.int32),
            pltpu.VMEM((_CHUNKS[-1][1],), jnp.int32),
            pltpu.VMEM((_CHUNK, _DE), jnp.float32),
            pltpu.VMEM_SHARED((_NP, _DE), jnp.float32),
        ],
    )(dst)


def _sc_edge_body(xs_hbm, xd_hbm, ep_hbm, src_hbm, dst_hbm, e_hbm, agg_hbm,
                  src_v, dst_v, dst_s, xs_v, xd_v, acc_v, res_v, eout_v, agg_sh,
                  sems):
    c = lax.axis_index("c")
    s = lax.axis_index("s")
    w = c * _NS + s

    def zrow(i, _):
        eout_v[i, :] = jnp.zeros((16,), jnp.float32)
        return 0
    lax.fori_loop(0, _RPT, zrow, 0)
    pltpu.sync_copy(eout_v.at[pl.ds(0, _RPT)], agg_sh.at[pl.ds(s * _RPT, _RPT)])
    plsc.subcore_barrier()

    def start(j):
        off, sz = _CHUNKS[j]
        b = j % 2
        base = w * _EW + off
        dv = dst_v[b] if sz == _CHUNK else dst_s
        pltpu.sync_copy(src_hbm.at[pl.ds(base, sz)], src_v[b].at[pl.ds(0, sz)])
        pltpu.sync_copy(dst_hbm.at[pl.ds(base, sz)], dv)
        cp1 = pltpu.async_copy(xs_hbm.at[src_v[b].at[pl.ds(0, sz)]],
                               xs_v[b].at[pl.ds(0, sz)], sems[2 * b])
        cp2 = pltpu.async_copy(xd_hbm.at[dv], xd_v[b].at[pl.ds(0, sz)],
                               sems[2 * b + 1])
        pltpu.sync_copy(ep_hbm.at[pl.ds(base * 16, sz * 16)],
                        acc_v[b].at[pl.ds(0, sz * 16)])
        return cp1, cp2, dv

    cur = start(0)
    for j in range(_NCHK):
        off, sz = _CHUNKS[j]
        b = j % 2
        base = w * _EW + off
        cp1, cp2, dv = cur
        nxt = start(j + 1) if j + 1 < _NCHK else None
        cp1.wait()
        cp2.wait()

        def erow(i, _):
            for k in range(16):
                row = i * 16 + k
                r = (acc_v[b][pl.ds(row * 16, 16)]
                     + xs_v[b][row, :] + xd_v[b][row, :])
                r = jnp.maximum(r, 0.0)
                eout_v[row, :] = r
                res_v[pl.ds(row * 16, 16)] = r
            return 0
        lax.fori_loop(0, sz // 16, erow, 0)

        pltpu.sync_copy(res_v.at[pl.ds(0, sz * 16)],
                        e_hbm.at[pl.ds(base * 16, sz * 16)])
        pltpu.sync_copy(eout_v.at[pl.ds(0, sz)], agg_sh.at[dv], add=True)
        cur = nxt

    plsc.subcore_barrier()
    pltpu.sync_copy(agg_sh.at[pl.ds(s * _RPT, _RPT)],
                    agg_hbm.at[c, pl.ds(s * _RPT, _RPT)])


def _sc_edge(xs, xd, ep1d, src, dst):
    return pl.kernel(
        _sc_edge_body,
        out_type=[
            jax.ShapeDtypeStruct((_E16,), jnp.float32),
            jax.ShapeDtypeStruct((_NC, _NP, _DE), jnp.float32),
        ],
        mesh=plsc.VectorSubcoreMesh(**_MESH),
        compiler_params=pltpu.CompilerParams(use_tc_tiling_on_sc=False),
        scratch_types=[
            (pltpu.VMEM((_CHUNK,), jnp.int32),) * 2,
            (pltpu.VMEM((_CHUNK,), jnp.int32),) * 2,
            pltpu.VMEM((_CHUNKS[-1][1],), jnp.int32),
            (pltpu.VMEM((_CHUNK, _DE), jnp.float32),) * 2,
            (pltpu.VMEM((_CHUNK, _DE), jnp.float32),) * 2,
            (pltpu.VMEM((_CHUNK * 16,), jnp.float32),) * 2,
            pltpu.VMEM((_CHUNK * 16,), jnp.float32),
            pltpu.VMEM((_CHUNK, _DE), jnp.float32),
            pltpu.VMEM_SHARED((_NP, _DE), jnp.float32),
            (pltpu.SemaphoreType.DMA,) * 4,
        ],
    )(xs, xd, ep1d, src, dst)


def kernel(node_feats, edge_index, edge_feats, glob_feats, batch,
           bn_ng, bn_nb, bn_eg, bn_eb, bn_gg, bn_gb,
           W_e, b_e, W_n, b_n, W_g, b_g):
    src = edge_index[0]
    dst = edge_index[1]
    bcol = batch[:, None]
    brow = batch.reshape(_N // 2000, 1, 2000)

    xs_sum, xs_sq = _colstats(node_feats, 2000)
    e8 = edge_feats.reshape(_E8, 128)
    es_sum128, es_sq128 = _colstats(e8, 4000)
    es_sum = jnp.sum(es_sum128.reshape(8, _DE), axis=0, keepdims=True)
    es_sq = jnp.sum(es_sq128.reshape(8, _DE), axis=0, keepdims=True)

    mx = xs_sum / _N
    vx = xs_sq / _N - mx * mx
    sx = bn_ng[None, :] / jnp.sqrt(vx + _EPS)
    tx = bn_nb[None, :] - mx * sx
    me = es_sum / _E
    ve = es_sq / _E - me * me
    se = bn_eg[None, :] / jnp.sqrt(ve + _EPS)
    te = bn_eb[None, :] - me * se

    Wes = [W_e[l, :_DN, :] for l in range(_NMP)]
    Wed = [W_e[l, _DN:2 * _DN, :] for l in range(_NMP)]
    Wee = [W_e[l, 2 * _DN:2 * _DN + _DE, :] for l in range(_NMP)]
    Weu = [W_e[l, 2 * _DN + _DE:, :] for l in range(_NMP)]
    Wnx = [W_n[l, :_DN, :] for l in range(_NMP)]
    Wne = [W_n[l, _DN:_DN + _DE, :] for l in range(_NMP)]
    Wnu = [W_n[l, _DN + _DE:, :] for l in range(_NMP)]
    Wgx = [W_g[l, :_DN, :] for l in range(_NMP)]
    Wgu = [W_g[l, _DN:, :] for l in range(_NMP)]

    Wes_eff = [sx.T * Wes[0]] + Wes[1:]
    Wed_eff = [sx.T * Wed[0]] + Wed[1:]
    Wee_eff = [se.T * Wee[0]] + Wee[1:]
    Wnx_eff = [sx.T * Wnx[0]] + Wnx[1:]
    eye8 = jnp.eye(8, dtype=jnp.float32)
    Wbig = [jnp.kron(eye8, w) for w in Wee_eff]

    u, ueff, unrow = _glob0(
        glob_feats, bn_gg[None, :], bn_gb[None, :], tx, te,
        Wes[0], Wed[0], Wee[0], Weu[0], b_e[0][None, :],
        Wnx[0], Wnu[0], b_n[0][None, :])

    cnt = _sc_prep(dst)
    c0 = cnt[0, :, :1]
    c1 = cnt[1, :, :1]

    x = node_feats
    for l in range(_NMP):
        has_next = l + 1 < _NMP
        if l == 0:
            xs, xd = _proj(x, Wes_eff[0], Wed_eff[0], ueff, bcol)
        ep1d = _ep8(e8, Wbig[l]).reshape(_E16)
        e1d, agg = _sc_edge(xs, xd, ep1d, src, dst)
        e8 = e1d.reshape(_E8, 128)
        x, naggsum, ncnt = _node(
            x, agg[0], agg[1], c0, c1, bcol, brow,
            Wnx_eff[l], Wne[l], unrow)
        if has_next:
            u, ueff, unrow = _glob(
                True, naggsum, ncnt, u, Wgx[l], Wgu[l], b_g[l][None, :],
                Weu[l + 1], b_e[l + 1][None, :], Wnu[l + 1], b_n[l + 1][None, :])
            xs, xd = _proj(x, Wes[l + 1], Wed[l + 1], ueff, bcol)
        else:
            u = _glob(False, naggsum, ncnt, u, Wgx[l], Wgu[l], b_g[l][None, :])
            u = u[0] if isinstance(u, (list, tuple)) else u
    e = e1d.reshape(_E, _DE)
    return (x, e, u)


# speedup vs baseline: 1.0500x; 1.0500x over previous
import functools

import jax
import jax.numpy as jnp
from jax import lax
from jax.experimental import pallas as pl
from jax.experimental.pallas import tpu as pltpu
from jax.experimental.pallas import tpu_sc as plsc

_N = 10000
_E = 320000
_E8 = _E // 8
_E16 = _E * 16
_DN = 128
_DE = 16
_DG = 32
_B = 16
_NMP = 3
_EPS = 1e-5

_NC = 2
_NS = 16
_NW = _NC * _NS
_EW = _E // _NW
_CHUNK = 1440
_CHUNKS = [(0, 1440), (1440, 1440), (2880, 1440), (4320, 1440),
           (5760, 1440), (7200, 1440), (8640, 1360)]
_NCHK = len(_CHUNKS)
_NP = 10240
_RPT = _NP // _NS


def _stats_body(x_ref, s_ref, ss_ref):
    @pl.when(pl.program_id(0) == 0)
    def _init():
        s_ref[...] = jnp.zeros_like(s_ref)
        ss_ref[...] = jnp.zeros_like(ss_ref)

    xb = x_ref[...]
    s_ref[...] += jnp.sum(xb, axis=0, keepdims=True)
    ss_ref[...] += jnp.sum(xb * xb, axis=0, keepdims=True)


def _colstats(x, bn):
    r, d = x.shape
    return pl.pallas_call(
        _stats_body,
        grid=(r // bn,),
        in_specs=[pl.BlockSpec((bn, d), lambda i: (i, 0))],
        out_specs=[pl.BlockSpec((1, d), lambda i: (0, 0))] * 2,
        out_shape=[jax.ShapeDtypeStruct((1, d), jnp.float32)] * 2,
    )(x)


def _proj_body(x_ref, ws_ref, wd_ref, ue_ref, bc_ref, xs_ref, xd_ref):
    xb = x_ref[...]
    bn = xb.shape[0]
    iot = lax.broadcasted_iota(jnp.int32, (bn, _B), 1)
    oh = (bc_ref[...] == iot).astype(jnp.float32)
    xs_ref[...] = (
        jnp.dot(xb, ws_ref[...], preferred_element_type=jnp.float32)
        + jnp.dot(oh, ue_ref[...], preferred_element_type=jnp.float32)
    )
    xd_ref[...] = jnp.dot(xb, wd_ref[...], preferred_element_type=jnp.float32)


def _proj(x, ws, wd, ueff, bcol, bn=2000):
    return pl.pallas_call(
        _proj_body,
        grid=(_N // bn,),
        in_specs=[
            pl.BlockSpec((bn, _DN), lambda i: (i, 0)),
            pl.BlockSpec((_DN, _DE), lambda i: (0, 0)),
            pl.BlockSpec((_DN, _DE), lambda i: (0, 0)),
            pl.BlockSpec((_B, _DE), lambda i: (0, 0)),
            pl.BlockSpec((bn, 1), lambda i: (i, 0)),
        ],
        out_specs=[pl.BlockSpec((bn, _DE), lambda i: (i, 0))] * 2,
        out_shape=[jax.ShapeDtypeStruct((_N, _DE), jnp.float32)] * 2,
    )(x, ws, wd, ueff, bcol)


def _projep_body(e8_ref, wbig_ref, x_ref, ws_ref, wd_ref, ue_ref, bc_ref,
                 ep_ref, xs_ref, xd_ref):
    ep_ref[...] = jnp.dot(e8_ref[...], wbig_ref[...],
                          preferred_element_type=jnp.float32)
    xb = x_ref[...]
    bn = xb.shape[0]
    iot = lax.broadcasted_iota(jnp.int32, (bn, _B), 1)
    oh = (bc_ref[...] == iot).astype(jnp.float32)
    xs_ref[...] = (
        jnp.dot(xb, ws_ref[...], preferred_element_type=jnp.float32)
        + jnp.dot(oh, ue_ref[...], preferred_element_type=jnp.float32)
    )
    xd_ref[...] = jnp.dot(xb, wd_ref[...], preferred_element_type=jnp.float32)


def _projep(e8, wbig, x, ws, wd, ueff, bcol, bn=2000, be=8000):
    return pl.pallas_call(
        _projep_body,
        grid=(_N // bn,),
        in_specs=[
            pl.BlockSpec((be, 128), lambda i: (i, 0)),
            pl.BlockSpec((128, 128), lambda i: (0, 0)),
            pl.BlockSpec((bn, _DN), lambda i: (i, 0)),
            pl.BlockSpec((_DN, _DE), lambda i: (0, 0)),
            pl.BlockSpec((_DN, _DE), lambda i: (0, 0)),
            pl.BlockSpec((_B, _DE), lambda i: (0, 0)),
            pl.BlockSpec((bn, 1), lambda i: (i, 0)),
        ],
        out_specs=[
            pl.BlockSpec((be, 128), lambda i: (i, 0)),
            pl.BlockSpec((bn, _DE), lambda i: (i, 0)),
            pl.BlockSpec((bn, _DE), lambda i: (i, 0)),
        ],
        out_shape=[
            jax.ShapeDtypeStruct((_E8, 128), jnp.float32),
            jax.ShapeDtypeStruct((_N, _DE), jnp.float32),
            jax.ShapeDtypeStruct((_N, _DE), jnp.float32),
        ],
    )(e8, wbig, x, ws, wd, ueff, bcol)


def _nodeglob_body(has_next, nsteps,
                   x_ref, a0_ref, a1_ref, c0_ref, c1_ref, bc_ref, br_ref,
                   wnx_ref, wne_ref, un_ref, u_ref, wgx_ref, wgu_ref, bg_ref,
                   *rest):
    if has_next:
        (weu_ref, be_ref, wnu_ref, bnn_ref,
         xn_ref, nagg_ref, ncnt_ref, unew_ref, ueff_ref, unrow_ref) = rest
    else:
        xn_ref, nagg_ref, ncnt_ref, unew_ref = rest
    bn = x_ref.shape[0]
    cnt = jnp.maximum(c0_ref[...] + c1_ref[...], 1.0)
    eagg = (a0_ref[...] + a1_ref[...]) / cnt
    iot = lax.broadcasted_iota(jnp.int32, (bn, _B), 1)
    oh = (bc_ref[...] == iot).astype(jnp.float32)
    xn = (
        jnp.dot(x_ref[...], wnx_ref[...], preferred_element_type=jnp.float32)
        + jnp.dot(eagg, wne_ref[...], preferred_element_type=jnp.float32)
        + jnp.dot(oh, un_ref[...], preferred_element_type=jnp.float32)
    )
    xn = jnp.maximum(xn, 0.0)
    xn_ref[...] = xn

    iot_t = lax.broadcasted_iota(jnp.int32, (_B, bn), 0)
    oh_t = (br_ref[...].reshape(1, bn) == iot_t).astype(jnp.float32)

    @pl.when(pl.program_id(0) == 0)
    def _init():
        nagg_ref[...] = jnp.zeros_like(nagg_ref)
        ncnt_ref[...] = jnp.zeros_like(ncnt_ref)

    nagg_ref[...] += jnp.dot(oh_t, xn, preferred_element_type=jnp.float32)
    ncnt_ref[...] += jnp.sum(oh_t, axis=1, keepdims=True)

    @pl.when(pl.program_id(0) == nsteps - 1)
    def _last():
        nagg = nagg_ref[...] / jnp.maximum(ncnt_ref[...], 1.0)
        unew = (
            jnp.dot(nagg, wgx_ref[...], preferred_element_type=jnp.float32)
            + jnp.dot(u_ref[...], wgu_ref[...],
                      preferred_element_type=jnp.float32)
            + bg_ref[...]
        )
        unew = jnp.maximum(unew, 0.0)
        unew_ref[...] = unew
        if has_next:
            ueff_ref[...] = (
                jnp.dot(unew, weu_ref[...], preferred_element_type=jnp.float32)
                + be_ref[...]
            )
            unrow_ref[...] = (
                jnp.dot(unew, wnu_ref[...], preferred_element_type=jnp.float32)
                + bnn_ref[...]
            )


def _nodeglob(has_next, x, a0, a1, c0, c1, bcol, brow, wnx, wne, unrow,
              u, wgx, wgu, bg, weu=None, be=None, wnu=None, bnn=None, bn=2000):
    nsteps = _N // bn
    full = lambda i: (0, 0)
    rows = lambda i: (i, 0)
    in_specs = [
        pl.BlockSpec((bn, _DN), rows),
        pl.BlockSpec((bn, _DE), rows),
        pl.BlockSpec((bn, _DE), rows),
        pl.BlockSpec((bn, 1), rows),
        pl.BlockSpec((bn, 1), rows),
        pl.BlockSpec((bn, 1), rows),
        pl.BlockSpec((1, 1, bn), lambda i: (i, 0, 0)),
        pl.BlockSpec((_DN, _DN), full),
        pl.BlockSpec((_DE, _DN), full),
        pl.BlockSpec((_B, _DN), full),
        pl.BlockSpec((_B, _DG), full),
        pl.BlockSpec((_DN, _DG), full),
        pl.BlockSpec((_DG, _DG), full),
        pl.BlockSpec((1, _DG), full),
    ]
    out_specs = [
        pl.BlockSpec((bn, _DN), rows),
        pl.BlockSpec((_B, _DN), full),
        pl.BlockSpec((_B, 1), full),
        pl.BlockSpec((_B, _DG), full),
    ]
    out_shape = [
        jax.ShapeDtypeStruct((_N, _DN), jnp.float32),
        jax.ShapeDtypeStruct((_B, _DN), jnp.float32),
        jax.ShapeDtypeStruct((_B, 1), jnp.float32),
        jax.ShapeDtypeStruct((_B, _DG), jnp.float32),
    ]
    args = [x, a0, a1, c0, c1, bcol, brow, wnx, wne, unrow, u, wgx, wgu, bg]
    if has_next:
        in_specs += [
            pl.BlockSpec((_DG, _DE), full),
            pl.BlockSpec((1, _DE), full),
            pl.BlockSpec((_DG, _DN), full),
            pl.BlockSpec((1, _DN), full),
        ]
        args += [weu, be, wnu, bnn]
        out_specs += [
            pl.BlockSpec((_B, _DE), full),
            pl.BlockSpec((_B, _DN), full),
        ]
        out_shape += [
            jax.ShapeDtypeStruct((_B, _DE), jnp.float32),
            jax.ShapeDtypeStruct((_B, _DN), jnp.float32),
        ]
    return pl.pallas_call(
        functools.partial(_nodeglob_body, has_next, nsteps),
        grid=(nsteps,),
        in_specs=in_specs,
        out_specs=out_specs,
        out_shape=out_shape,
    )(*args)


def _glob0_body(g_ref, gg_ref, gb_ref, tx_ref, te_ref,
                wes_ref, wed_ref, wee_ref, weu_ref, be_ref,
                wnx_ref, wnu_ref, bn_ref,
                u0_ref, ueff_ref, unrow_ref):
    g = g_ref[...]
    m = jnp.mean(g, axis=0, keepdims=True)
    v = jnp.mean(g * g, axis=0, keepdims=True) - m * m
    ub = (g - m) * lax.rsqrt(v + _EPS) * gg_ref[...] + gb_ref[...]
    u0_ref[...] = ub
    tx = tx_ref[...]
    te = te_ref[...]
    ueff_ref[...] = (
        jnp.dot(ub, weu_ref[...], preferred_element_type=jnp.float32)
        + be_ref[...]
        + jnp.dot(te, wee_ref[...], preferred_element_type=jnp.float32)
        + jnp.dot(tx, wes_ref[...], preferred_element_type=jnp.float32)
        + jnp.dot(tx, wed_ref[...], preferred_element_type=jnp.float32)
    )
    unrow_ref[...] = (
        jnp.dot(ub, wnu_ref[...], preferred_element_type=jnp.float32)
        + bn_ref[...]
        + jnp.dot(tx, wnx_ref[...], preferred_element_type=jnp.float32)
    )


def _glob0(g, gg, gb, tx, te, wes, wed, wee, weu, be, wnx, wnu, bnn):
    return pl.pallas_call(
        _glob0_body,
        out_shape=[
            jax.ShapeDtypeStruct((_B, _DG), jnp.float32),
            jax.ShapeDtypeStruct((_B, _DE), jnp.float32),
            jax.ShapeDtypeStruct((_B, _DN), jnp.float32),
        ],
    )(g, gg, gb, tx, te, wes, wed, wee, weu, be, wnx, wnu, bnn)


_MESH = dict(core_axis_name="c", subcore_axis_name="s",
             num_cores=_NC, num_subcores=_NS)


def _sc_prep_body(dst_hbm, cnt_hbm, dst_v, dst_s, ones_v, cnt_sh):
    c = lax.axis_index("c")
    s = lax.axis_index("s")
    w = c * _NS + s

    def zrow(i, _):
        ones_v[i, :] = jnp.zeros((16,), jnp.float32)
        return 0
    lax.fori_loop(0, _RPT, zrow, 0)
    pltpu.sync_copy(ones_v.at[pl.ds(0, _RPT)], cnt_sh.at[pl.ds(s * _RPT, _RPT)])
    plsc.subcore_barrier()

    def orow(i, _):
        ones_v[i, :] = jnp.ones((16,), jnp.float32)
        return 0
    lax.fori_loop(0, _CHUNK, orow, 0)

    for off, sz in _CHUNKS:
        base = w * _EW + off
        dv = dst_v if sz == _CHUNK else dst_s
        pltpu.sync_copy(dst_hbm.at[pl.ds(base, sz)], dv)
        pltpu.sync_copy(ones_v.at[pl.ds(0, sz)], cnt_sh.at[dv], add=True)

    plsc.subcore_barrier()
    pltpu.sync_copy(cnt_sh.at[pl.ds(s * _RPT, _RPT)],
                    cnt_hbm.at[c, pl.ds(s * _RPT, _RPT)])


def _sc_prep(dst):
    return pl.kernel(
        _sc_prep_body,
        out_type=jax.ShapeDtypeStruct((_NC, _NP, _DE), jnp.float32),
        mesh=plsc.VectorSubcoreMesh(**_MESH),
        compiler_params=pltpu.CompilerParams(use_tc_tiling_on_sc=False),
        scratch_types=[
            pltpu.VMEM((_CHUNK,), jnp.int32),
            pltpu.VMEM((_CHUNKS[-1][1],), jnp.int32),
            pltpu.VMEM((_CHUNK, _DE), jnp.float32),
            pltpu.VMEM_SHARED((_NP, _DE), jnp.float32),
        ],
    )(dst)


def _sc_edge_body(xs_hbm, xd_hbm, ep_hbm, src_hbm, dst_hbm, e_hbm, agg_hbm,
                  src_v, dst_v, dst_s, xs_v, xd_v, acc_v, res_v, eout_v, agg_sh,
                  sems):
    c = lax.axis_index("c")
    s = lax.axis_index("s")
    w = c * _NS + s

    def zrow(i, _):
        eout_v[i, :] = jnp.zeros((16,), jnp.float32)
        return 0
    lax.fori_loop(0, _RPT, zrow, 0)
    pltpu.sync_copy(eout_v.at[pl.ds(0, _RPT)], agg_sh.at[pl.ds(s * _RPT, _RPT)])
    plsc.subcore_barrier()

    for off, sz in _CHUNKS:
        base = w * _EW + off
        dv = dst_v[0] if sz == _CHUNK else dst_s
        pltpu.sync_copy(src_hbm.at[pl.ds(base, sz)], src_v[0].at[pl.ds(0, sz)])
        pltpu.sync_copy(dst_hbm.at[pl.ds(base, sz)], dv)
        cp1 = pltpu.async_copy(xs_hbm.at[src_v[0].at[pl.ds(0, sz)]],
                               xs_v[0].at[pl.ds(0, sz)], sems[0])
        cp2 = pltpu.async_copy(xd_hbm.at[dv], xd_v[0].at[pl.ds(0, sz)], sems[1])
        pltpu.sync_copy(ep_hbm.at[pl.ds(base * 16, sz * 16)],
                        acc_v[0].at[pl.ds(0, sz * 16)])
        cp1.wait()
        cp2.wait()

        def erow(i, _):
            for k in range(16):
                row = i * 16 + k
                r = (acc_v[0][pl.ds(row * 16, 16)]
                     + xs_v[0][row, :] + xd_v[0][row, :])
                r = jnp.maximum(r, 0.0)
                eout_v[row, :] = r
                res_v[pl.ds(row * 16, 16)] = r
            return 0
        lax.fori_loop(0, sz // 16, erow, 0)

        pltpu.sync_copy(res_v.at[pl.ds(0, sz * 16)],
                        e_hbm.at[pl.ds(base * 16, sz * 16)])
        pltpu.sync_copy(eout_v.at[pl.ds(0, sz)], agg_sh.at[dv], add=True)

    plsc.subcore_barrier()
    pltpu.sync_copy(agg_sh.at[pl.ds(s * _RPT, _RPT)],
                    agg_hbm.at[c, pl.ds(s * _RPT, _RPT)])


def _sc_edge(xs, xd, ep1d, src, dst):
    return pl.kernel(
        _sc_edge_body,
        out_type=[
            jax.ShapeDtypeStruct((_E16,), jnp.float32),
            jax.ShapeDtypeStruct((_NC, _NP, _DE), jnp.float32),
        ],
        mesh=plsc.VectorSubcoreMesh(**_MESH),
        compiler_params=pltpu.CompilerParams(use_tc_tiling_on_sc=False),
        scratch_types=[
            (pltpu.VMEM((_CHUNK,), jnp.int32),) * 1,
            (pltpu.VMEM((_CHUNK,), jnp.int32),) * 1,
            pltpu.VMEM((_CHUNKS[-1][1],), jnp.int32),
            (pltpu.VMEM((_CHUNK, _DE), jnp.float32),) * 1,
            (pltpu.VMEM((_CHUNK, _DE), jnp.float32),) * 1,
            (pltpu.VMEM((_CHUNK * 16,), jnp.float32),) * 1,
            pltpu.VMEM((_CHUNK * 16,), jnp.float32),
            pltpu.VMEM((_CHUNK, _DE), jnp.float32),
            pltpu.VMEM_SHARED((_NP, _DE), jnp.float32),
            (pltpu.SemaphoreType.DMA,) * 2,
        ],
    )(xs, xd, ep1d, src, dst)


def kernel(node_feats, edge_index, edge_feats, glob_feats, batch,
           bn_ng, bn_nb, bn_eg, bn_eb, bn_gg, bn_gb,
           W_e, b_e, W_n, b_n, W_g, b_g):
    src = edge_index[0]
    dst = edge_index[1]
    bcol = batch[:, None]
    brow = batch.reshape(_N // 2000, 1, 2000)

    xs_sum, xs_sq = _colstats(node_feats, 2000)
    e8 = edge_feats.reshape(_E8, 128)
    es_sum128, es_sq128 = _colstats(e8, 4000)
    es_sum = jnp.sum(es_sum128.reshape(8, _DE), axis=0, keepdims=True)
    es_sq = jnp.sum(es_sq128.reshape(8, _DE), axis=0, keepdims=True)

    mx = xs_sum / _N
    vx = xs_sq / _N - mx * mx
    sx = bn_ng[None, :] / jnp.sqrt(vx + _EPS)
    tx = bn_nb[None, :] - mx * sx
    me = es_sum / _E
    ve = es_sq / _E - me * me
    se = bn_eg[None, :] / jnp.sqrt(ve + _EPS)
    te = bn_eb[None, :] - me * se

    Wes = [W_e[l, :_DN, :] for l in range(_NMP)]
    Wed = [W_e[l, _DN:2 * _DN, :] for l in range(_NMP)]
    Wee = [W_e[l, 2 * _DN:2 * _DN + _DE, :] for l in range(_NMP)]
    Weu = [W_e[l, 2 * _DN + _DE:, :] for l in range(_NMP)]
    Wnx = [W_n[l, :_DN, :] for l in range(_NMP)]
    Wne = [W_n[l, _DN:_DN + _DE, :] for l in range(_NMP)]
    Wnu = [W_n[l, _DN + _DE:, :] for l in range(_NMP)]
    Wgx = [W_g[l, :_DN, :] for l in range(_NMP)]
    Wgu = [W_g[l, _DN:, :] for l in range(_NMP)]

    Wes_eff = [sx.T * Wes[0]] + Wes[1:]
    Wed_eff = [sx.T * Wed[0]] + Wed[1:]
    Wee_eff = [se.T * Wee[0]] + Wee[1:]
    Wnx_eff = [sx.T * Wnx[0]] + Wnx[1:]
    eye8 = jnp.eye(8, dtype=jnp.float32)
    Wbig = [jnp.kron(eye8, w) for w in Wee_eff]

    u, ueff, unrow = _glob0(
        glob_feats, bn_gg[None, :], bn_gb[None, :], tx, te,
        Wes[0], Wed[0], Wee[0], Weu[0], b_e[0][None, :],
        Wnx[0], Wnu[0], b_n[0][None, :])

    cnt = _sc_prep(dst)
    c0 = cnt[0, :, :1]
    c1 = cnt[1, :, :1]

    x = node_feats
    ep8a, xs, xd = _projep(e8, Wbig[0], x, Wes_eff[0], Wed_eff[0], ueff, bcol)
    for l in range(_NMP):
        has_next = l + 1 < _NMP
        e1d, agg = _sc_edge(xs, xd, ep8a.reshape(_E16), src, dst)
        e8 = e1d.reshape(_E8, 128)
        if has_next:
            x, naggsum, ncnt, u, ueff, unrow = _nodeglob(
                True, x, agg[0], agg[1], c0, c1, bcol, brow,
                Wnx_eff[l], Wne[l], unrow, u, Wgx[l], Wgu[l], b_g[l][None, :],
                Weu[l + 1], b_e[l + 1][None, :], Wnu[l + 1],
                b_n[l + 1][None, :])
            ep8a, xs, xd = _projep(e8, Wbig[l + 1], x, Wes[l + 1], Wed[l + 1],
                                   ueff, bcol)
        else:
            x, naggsum, ncnt, u = _nodeglob(
                False, x, agg[0], agg[1], c0, c1, bcol, brow,
                Wnx_eff[l], Wne[l], unrow, u, Wgx[l], Wgu[l], b_g[l][None, :])
    e = e1d.reshape(_E, _DE)
    return (x, e, u)
